# Initial kernel scaffold; baseline (speedup 1.0000x reference)
import jax
import jax.numpy as jnp
from jax.experimental import pallas as pl


def kernel(edge_index, N, y, emb, W1, att_src1, att_dst1, b1, Wm, bm, W2, att_src2, att_dst2, b2, Wm1, bm1, Wout, bout):
    raise NotImplementedError("write your pallas kernel here")


# speedup vs baseline: 1.0492x; 1.0492x over previous
import functools

import jax
import jax.numpy as jnp
from jax.experimental import pallas as pl
from jax.experimental.pallas import tpu as pltpu

D_MODEL = 128
D_HIDDEN = 128
HEADS = 8
N_NODES = 10000
N_TRG = 256

VOCAB_PAD = 50176
VBLK = 3584


def _layernorm(x, eps=1e-5):
    mu = jnp.mean(x, axis=-1, keepdims=True)
    var = jnp.mean((x - mu) ** 2, axis=-1, keepdims=True)
    return (x - mu) / jnp.sqrt(var + eps)


def _gat_conv(x, edge_index, W, att_src, att_dst, bias):
    n = x.shape[0]
    loop = jnp.arange(n, dtype=edge_index.dtype)
    src = jnp.concatenate([edge_index[0], loop])
    dst = jnp.concatenate([edge_index[1], loop])
    h = (x @ W).reshape(n, HEADS, -1)
    a_src = jnp.sum(h * att_src[None], axis=-1)
    a_dst = jnp.sum(h * att_dst[None], axis=-1)
    e = a_src[src] + a_dst[dst]
    e = jnp.where(e > 0, e, 0.2 * e)
    ex = jnp.exp(e)
    s = jax.ops.segment_sum(ex, dst, num_segments=n)
    alpha = ex / (s[dst] + 1e-16)
    out = jax.ops.segment_sum(h[src] * alpha[:, :, None], dst, num_segments=n)
    return jnp.mean(out, axis=1) + bias


def _gelu_exact(x):
    return 0.5 * x * (1.0 + jax.lax.erf(x * (2.0 ** -0.5)))


def _final_proj_body(trg_ref, wm1_ref, bm1_ref, wout_ref, bout_ref, out_ref):
    t = trg_ref[...] @ wm1_ref[...] + bm1_ref[...]
    t = _gelu_exact(t)
    out_ref[...] = t @ wout_ref[...] + bout_ref[...]


def _final_proj(trg, Wm1, bm1, Wout, bout):
    wout_p = jnp.zeros((D_HIDDEN, VOCAB_PAD), jnp.float32).at[:, :50000].set(Wout)
    bout_p = jnp.zeros((1, VOCAB_PAD), jnp.float32).at[0, :50000].set(bout)
    grid = VOCAB_PAD // VBLK
    out = pl.pallas_call(
        _final_proj_body,
        grid=(grid,),
        in_specs=[
            pl.BlockSpec((N_TRG, D_HIDDEN), lambda j: (0, 0)),
            pl.BlockSpec((D_HIDDEN, D_HIDDEN), lambda j: (0, 0)),
            pl.BlockSpec((1, D_HIDDEN), lambda j: (0, 0)),
            pl.BlockSpec((D_HIDDEN, VBLK), lambda j: (0, j)),
            pl.BlockSpec((1, VBLK), lambda j: (0, j)),
        ],
        out_specs=pl.BlockSpec((N_TRG, VBLK), lambda j: (0, j)),
        out_shape=jax.ShapeDtypeStruct((N_TRG, VOCAB_PAD), jnp.float32),
    )(trg, Wm1, bm1.reshape(1, -1), wout_p, bout_p)
    return out[:, :50000]


@jax.jit
def kernel(edge_index, N, y, emb, W1, att_src1, att_dst1, b1, Wm, bm,
           W2, att_src2, att_dst2, b2, Wm1, bm1, Wout, bout):
    x = _layernorm(emb[y])
    x = _gat_conv(x, edge_index, W1, att_src1, att_dst1, b1)
    x = jnp.where(x > 0, x, 0.15 * x)
    x = jax.nn.gelu(x @ Wm + bm, approximate=False)
    x = _gat_conv(x, edge_index, W2, att_src2, att_dst2, b2)
    trg = x[N]
    return _final_proj(trg, Wm1, bm1, Wout, bout)


# speedup vs baseline: 7.1104x; 6.7767x over previous
import dataclasses
import functools

import jax
import jax.numpy as jnp
from jax import lax
from jax.experimental import pallas as pl
from jax.experimental.pallas import tpu as pltpu
from jax.experimental.pallas import tpu_sc as plsc

D_MODEL = 128
D_HIDDEN = 128
HEADS = 8
N_NODES = 10000
N_TRG = 256
N_EDGES = 320000

NW = 32
EP = 331776
NE_REAL = N_EDGES + N_NODES
TPW = EP // NW
C1 = 96
C2 = 32
NP = 10240
ROWS_PT = NP // 16
SH_ROWS = 10112
SPT = SH_ROWS // 16
DH = HEADS * D_HIDDEN
DEXT = DH + 128

VOCAB_PAD = 50176
VBLK = 3584

_mesh = plsc.VectorSubcoreMesh(core_axis_name="c", subcore_axis_name="s")

_sc_params = pltpu.CompilerParams()
if "needs_layout_passes" in pltpu.CompilerParams.__dataclass_fields__:
    _sc_params = dataclasses.replace(_sc_params, needs_layout_passes=False)


_HI = jax.lax.Precision.HIGHEST


def _dot(a, b):
    return jnp.dot(a.astype(jnp.bfloat16), b.astype(jnp.bfloat16),
                   preferred_element_type=jnp.float32)


def _dot_hi(a, b):
    return jnp.dot(a, b, preferred_element_type=jnp.float32, precision=_HI)


def _gelu_exact(x):
    return 0.5 * x * (1.0 + jax.lax.erf(x * (2.0 ** -0.5)))


def _sc_gather(table, idx, D):
    B = idx.shape[0]
    b_per_w = B // NW

    @functools.partial(
        pl.kernel, mesh=_mesh,
        out_type=jax.ShapeDtypeStruct((B, D), jnp.float32),
        scratch_types=[
            pltpu.VMEM((b_per_w,), jnp.int32),
            pltpu.VMEM((b_per_w, D), jnp.float32),
            pltpu.SemaphoreType.DMA,
        ],
    )
    def k(table_hbm, idx_hbm, out_hbm, idx_v, rows_v, sem):
        wid = lax.axis_index("s") * 2 + lax.axis_index("c")
        base = wid * b_per_w
        pltpu.sync_copy(idx_hbm.at[pl.ds(base, b_per_w)], idx_v)
        pltpu.async_copy(table_hbm.at[idx_v], rows_v, sem).wait()
        pltpu.sync_copy(rows_v, out_hbm.at[pl.ds(base, b_per_w)])

    return k(table, idx)


def _sc_edge_logits(srcx, dstx, asrcp, adstp):
    @functools.partial(
        pl.kernel, mesh=_mesh,
        out_type=jax.ShapeDtypeStruct((2 * NP, 128), jnp.float32),
        scratch_types=[
            pltpu.VMEM((C1,), jnp.int32),
            pltpu.VMEM((C1,), jnp.int32),
            pltpu.VMEM((C1, 128), jnp.float32),
            pltpu.VMEM((C1, 128), jnp.float32),
            pltpu.VMEM((C1, 128), jnp.float32),
            pltpu.VMEM_SHARED((SH_ROWS, 128), jnp.float32),
            pltpu.SemaphoreType.DMA,
        ],
    )
    def k(src_hbm, dst_hbm, asrc_hbm, adst_hbm, spart_hbm,
          sidx, didx, asb, adb, exb, s_sh, sem):
        cid = lax.axis_index("c")
        sid = lax.axis_index("s")
        wid = sid * 2 + cid

        @pl.loop(0, C1)
        def _(i):
            for j in range(8):
                exb[i, pl.ds(j * 16, 16)] = jnp.zeros((16,), jnp.float32)

        @pl.loop(0, SPT // 8)
        def _(j):
            pltpu.sync_copy(exb.at[pl.ds(0, 8)],
                            s_sh.at[pl.ds(sid * SPT + j * 8, 8)])

        plsc.subcore_barrier()

        base = wid * TPW

        @pl.loop(0, TPW // C1)
        def _(c):
            off = base + c * C1
            pltpu.sync_copy(src_hbm.at[pl.ds(off, C1)], sidx)
            pltpu.sync_copy(dst_hbm.at[pl.ds(off, C1)], didx)
            pltpu.async_copy(asrc_hbm.at[sidx], asb, sem).wait()
            pltpu.async_copy(adst_hbm.at[didx], adb, sem).wait()

            @pl.loop(0, C1)
            def _(e):
                z = asb[e, pl.ds(0, 16)] + adb[e, pl.ds(0, 16)]
                z = jnp.maximum(z, 0.2 * z)
                ex = jnp.exp(z)
                lim = jnp.full((16,), NE_REAL - (off + e), jnp.int32)
                mask = jnp.clip(lim, 0, 1).astype(jnp.float32)
                exb[e, pl.ds(0, 16)] = ex * mask

            pltpu.sync_copy(exb, s_sh.at[didx], add=True)

        plsc.subcore_barrier()
        pltpu.sync_copy(
            s_sh.at[pl.ds(sid * SPT, SPT)],
            spart_hbm.at[pl.ds(cid * NP + sid * SPT, SPT)],
        )

    return k(srcx, dstx, asrcp, adstp)


def _sc_edge_aggregate(srcx, dstx, hext, dstpack):
    @functools.partial(
        pl.kernel, mesh=_mesh, compiler_params=_sc_params,
        out_type=jax.ShapeDtypeStruct((2 * NP, 128), jnp.float32),
        scratch_types=[
            pltpu.VMEM((C2,), jnp.int32),
            pltpu.VMEM((C2,), jnp.int32),
            pltpu.VMEM((C2, DEXT), jnp.float32),
            pltpu.VMEM((C2, 128), jnp.float32),
            pltpu.VMEM((C2, 128), jnp.float32),
            pltpu.VMEM((8, 128), jnp.float32),
            pltpu.VMEM((16,), jnp.float32),
            pltpu.VMEM_SHARED((SH_ROWS, 128), jnp.float32),
            pltpu.SemaphoreType.DMA,
        ],
    )
    def k(src_hbm, dst_hbm, hext_hbm, dpk_hbm, opart_hbm,
          sidx, didx, hb, dpb, cb, zb, wv, o_sh, sem):
        cid = lax.axis_index("c")
        sid = lax.axis_index("s")
        wid = sid * 2 + cid

        @pl.loop(0, 8)
        def _(i):
            for j in range(8):
                zb[i, pl.ds(j * 16, 16)] = jnp.zeros((16,), jnp.float32)

        @pl.loop(0, SPT // 8)
        def _(j):
            pltpu.sync_copy(zb, o_sh.at[pl.ds(sid * SPT + j * 8, 8)])

        plsc.subcore_barrier()

        base = wid * TPW

        @pl.loop(0, TPW // C2)
        def _(c):
            off = base + c * C2
            pltpu.sync_copy(src_hbm.at[pl.ds(off, C2)], sidx)
            pltpu.sync_copy(dst_hbm.at[pl.ds(off, C2)], didx)
            pltpu.async_copy(hext_hbm.at[sidx], hb, sem).wait()
            pltpu.async_copy(dpk_hbm.at[didx], dpb, sem).wait()

            @pl.loop(0, C2)
            def _(e):
                z = hb[e, pl.ds(DH, 16)] + dpb[e, pl.ds(0, 16)]
                z = jnp.maximum(z, 0.2 * z)
                ex = jnp.exp(z)
                lim = jnp.full((16,), NE_REAL - (off + e), jnp.int32)
                mask = jnp.clip(lim, 0, 1).astype(jnp.float32)
                wv[...] = ex * mask * dpb[e, pl.ds(16, 16)]
                for hh in range(HEADS):
                    wb = plsc.load_gather(
                        wv, [jnp.full((16,), 8 + hh, jnp.int32)])
                    for j in range(8):
                        hs = hb[e, pl.ds(hh * 128 + j * 16, 16)]
                        if hh == 0:
                            cb[e, pl.ds(j * 16, 16)] = wb * hs
                        else:
                            cb[e, pl.ds(j * 16, 16)] += wb * hs

            pltpu.sync_copy(cb, o_sh.at[didx], add=True)

        plsc.subcore_barrier()
        pltpu.sync_copy(
            o_sh.at[pl.ds(sid * SPT, SPT)],
            opart_hbm.at[pl.ds(cid * NP + sid * SPT, SPT)],
        )

    return k(srcx, dstx, hext, dstpack)


def _d1_body(xg_ref, w1_ref, as_ref, ad_ref, hext_ref, asp_ref, adp_ref):
    x = xg_ref[...]
    mu = jnp.mean(x, axis=-1, keepdims=True)
    var = jnp.mean((x - mu) ** 2, axis=-1, keepdims=True)
    x = (x - mu) / jnp.sqrt(var + 1e-5)
    h = _dot(x, w1_ref[...])
    a_s = _dot_hi(h, as_ref[...])
    a_d = _dot_hi(h, ad_ref[...])
    hext_ref[:, :DH] = h
    hext_ref[:, DH:] = a_s
    asp_ref[...] = a_s
    adp_ref[...] = a_d


def _d1(xg, W1, Asrc, Adst):
    R = 512
    return pl.pallas_call(
        _d1_body,
        grid=(NP // R,),
        in_specs=[
            pl.BlockSpec((R, D_MODEL), lambda i: (i, 0)),
            pl.BlockSpec((D_MODEL, DH), lambda i: (0, 0)),
            pl.BlockSpec((DH, 128), lambda i: (0, 0)),
            pl.BlockSpec((DH, 128), lambda i: (0, 0)),
        ],
        out_specs=[
            pl.BlockSpec((R, DEXT), lambda i: (i, 0)),
            pl.BlockSpec((R, 128), lambda i: (i, 0)),
            pl.BlockSpec((R, 128), lambda i: (i, 0)),
        ],
        out_shape=[
            jax.ShapeDtypeStruct((NP, DEXT), jnp.float32),
            jax.ShapeDtypeStruct((NP, 128), jnp.float32),
            jax.ShapeDtypeStruct((NP, 128), jnp.float32),
        ],
    )(xg, W1, Asrc, Adst)


def _pack_body(adp_ref, s0_ref, s1_ref, out_ref):
    s = s0_ref[...] + s1_ref[...]
    r = 1.0 / (8.0 * (s + 1e-16))
    out_ref[:, :16] = adp_ref[:, :16]
    out_ref[:, 16:32] = r[:, :16]
    out_ref[:, 32:] = jnp.zeros_like(out_ref[:, 32:])


def _pack_dst(adp, s0, s1):
    R = 512
    return pl.pallas_call(
        _pack_body,
        grid=(NP // R,),
        in_specs=[
            pl.BlockSpec((R, 128), lambda i: (i, 0)),
            pl.BlockSpec((R, 128), lambda i: (i, 0)),
            pl.BlockSpec((R, 128), lambda i: (i, 0)),
        ],
        out_specs=pl.BlockSpec((R, 128), lambda i: (i, 0)),
        out_shape=jax.ShapeDtypeStruct((NP, 128), jnp.float32),
    )(adp, s0, s1)


def _d2_body(o0_ref, o1_ref, b1_ref, wm_ref, bm_ref, w2_ref, as_ref, ad_ref,
             hext_ref, asp_ref, adp_ref):
    g = o0_ref[...] + o1_ref[...] + b1_ref[...]
    g = jnp.where(g > 0, g, 0.15 * g)
    t = _dot(g, wm_ref[...]) + bm_ref[...]
    t = _gelu_exact(t)
    h = _dot(t, w2_ref[...])
    a_s = _dot_hi(h, as_ref[...])
    a_d = _dot_hi(h, ad_ref[...])
    hext_ref[:, :DH] = h
    hext_ref[:, DH:] = a_s
    asp_ref[...] = a_s
    adp_ref[...] = a_d


def _d2(o0, o1, b1, Wm, bm, W2, Asrc, Adst):
    R = 512
    return pl.pallas_call(
        _d2_body,
        grid=(NP // R,),
        in_specs=[
            pl.BlockSpec((R, D_HIDDEN), lambda i: (i, 0)),
            pl.BlockSpec((R, D_HIDDEN), lambda i: (i, 0)),
            pl.BlockSpec((1, D_HIDDEN), lambda i: (0, 0)),
            pl.BlockSpec((D_HIDDEN, D_HIDDEN), lambda i: (0, 0)),
            pl.BlockSpec((1, D_HIDDEN), lambda i: (0, 0)),
            pl.BlockSpec((D_HIDDEN, DH), lambda i: (0, 0)),
            pl.BlockSpec((DH, 128), lambda i: (0, 0)),
            pl.BlockSpec((DH, 128), lambda i: (0, 0)),
        ],
        out_specs=[
            pl.BlockSpec((R, DEXT), lambda i: (i, 0)),
            pl.BlockSpec((R, 128), lambda i: (i, 0)),
            pl.BlockSpec((R, 128), lambda i: (i, 0)),
        ],
        out_shape=[
            jax.ShapeDtypeStruct((NP, DEXT), jnp.float32),
            jax.ShapeDtypeStruct((NP, 128), jnp.float32),
            jax.ShapeDtypeStruct((NP, 128), jnp.float32),
        ],
    )(o0, o1, b1.reshape(1, -1), Wm, bm.reshape(1, -1), W2, Asrc, Adst)


def _comb_body(o0_ref, o1_ref, b2_ref, y_ref):
    y_ref[...] = o0_ref[...] + o1_ref[...] + b2_ref[...]


def _combine2(o0, o1, b2):
    R = 512
    return pl.pallas_call(
        _comb_body,
        grid=(NP // R,),
        in_specs=[
            pl.BlockSpec((R, D_HIDDEN), lambda i: (i, 0)),
            pl.BlockSpec((R, D_HIDDEN), lambda i: (i, 0)),
            pl.BlockSpec((1, D_HIDDEN), lambda i: (0, 0)),
        ],
        out_specs=pl.BlockSpec((R, D_HIDDEN), lambda i: (i, 0)),
        out_shape=jax.ShapeDtypeStruct((NP, D_HIDDEN), jnp.float32),
    )(o0, o1, b2.reshape(1, -1))


def _final_proj_body(trg_ref, wm1_ref, bm1_ref, wout_ref, bout_ref, out_ref):
    t = _dot(trg_ref[...], wm1_ref[...]) + bm1_ref[...]
    t = _gelu_exact(t)
    out_ref[...] = _dot(t, wout_ref[...]) + bout_ref[...]


def _final_proj(trg, Wm1, bm1, Wout, bout):
    wout_p = jnp.zeros((D_HIDDEN, VOCAB_PAD), jnp.float32).at[:, :50000].set(Wout)
    bout_p = jnp.zeros((1, VOCAB_PAD), jnp.float32).at[0, :50000].set(bout)
    out = pl.pallas_call(
        _final_proj_body,
        grid=(VOCAB_PAD // VBLK,),
        in_specs=[
            pl.BlockSpec((N_TRG, D_HIDDEN), lambda j: (0, 0)),
            pl.BlockSpec((D_HIDDEN, D_HIDDEN), lambda j: (0, 0)),
            pl.BlockSpec((1, D_HIDDEN), lambda j: (0, 0)),
            pl.BlockSpec((D_HIDDEN, VBLK), lambda j: (0, j)),
            pl.BlockSpec((1, VBLK), lambda j: (0, j)),
        ],
        out_specs=pl.BlockSpec((N_TRG, VBLK), lambda j: (0, j)),
        out_shape=jax.ShapeDtypeStruct((N_TRG, VOCAB_PAD), jnp.float32),
    )(trg, Wm1, bm1.reshape(1, -1), wout_p, bout_p)
    return out[:, :50000]


def _att_proj(att):
    a = jnp.einsum("hc,hk->hck", att, jnp.eye(HEADS, dtype=att.dtype))
    a = a.reshape(DH, HEADS)
    a = jnp.concatenate([a, a], axis=-1)
    return jnp.pad(a, ((0, 0), (0, 128 - 2 * HEADS)))


@jax.jit
def kernel(edge_index, N, y, emb, W1, att_src1, att_dst1, b1, Wm, bm,
           W2, att_src2, att_dst2, b2, Wm1, bm1, Wout, bout):
    loop = jnp.arange(N_NODES, dtype=jnp.int32)
    padz = jnp.zeros((EP - NE_REAL,), jnp.int32)
    srcx = jnp.concatenate([edge_index[0], loop, padz])
    dstx = jnp.concatenate([edge_index[1], loop, padz])

    y_pad = jnp.concatenate([y, jnp.zeros((NP - N_NODES,), jnp.int32)])
    xg = _sc_gather(emb, y_pad, D_MODEL)

    hext1, asp1, adp1 = _d1(xg, W1, _att_proj(att_src1), _att_proj(att_dst1))
    spart1 = _sc_edge_logits(srcx, dstx, asp1, adp1)
    dpk1 = _pack_dst(adp1, spart1[:NP], spart1[NP:])
    opart1 = _sc_edge_aggregate(srcx, dstx, hext1, dpk1)

    hext2, asp2, adp2 = _d2(opart1[:NP], opart1[NP:], b1, Wm, bm,
                            W2, _att_proj(att_src2), _att_proj(att_dst2))
    spart2 = _sc_edge_logits(srcx, dstx, asp2, adp2)
    dpk2 = _pack_dst(adp2, spart2[:NP], spart2[NP:])
    opart2 = _sc_edge_aggregate(srcx, dstx, hext2, dpk2)

    y2 = _combine2(opart2[:NP], opart2[NP:], b2)

    trg = _sc_gather(y2, N, D_HIDDEN)
    return _final_proj(trg, Wm1, bm1, Wout, bout)
